# baseline (speedup 1.0000x reference)
import functools

import numpy as np
import jax
import jax.numpy as jnp
from jax import lax
from jax.experimental import pallas as pl
from jax.experimental.pallas import tpu as pltpu
from jax.experimental.pallas import tpu_sc as plsc

_NUM_ABS = 26
_ROW_BLOCK = 4000
_CH = 640
_PERW = 3200


def _rotl32(x, r):
    r = np.uint32(r)
    return ((x << r) | (x >> (np.uint32(32) - r))).astype(np.uint32)


def _threefry2x32(k0, k1, x0, x1):
    x0 = x0.astype(np.uint32)
    x1 = x1.astype(np.uint32)
    ks0 = np.uint32(k0)
    ks1 = np.uint32(k1)
    ks2 = np.uint32(ks0 ^ ks1 ^ np.uint32(0x1BD11BDA))
    ks = (ks0, ks1, ks2)
    rotations = ((13, 15, 26, 6), (17, 29, 16, 24))
    x0 = (x0 + ks0).astype(np.uint32)
    x1 = (x1 + ks1).astype(np.uint32)
    for i in range(5):
        for r in rotations[i % 2]:
            x0 = (x0 + x1).astype(np.uint32)
            x1 = _rotl32(x1, r)
            x1 = (x1 ^ x0).astype(np.uint32)
        x0 = (x0 + ks[(i + 1) % 3]).astype(np.uint32)
        x1 = (x1 + ks[(i + 2) % 3] + np.uint32(i + 1)).astype(np.uint32)
    return x0, x1


@functools.lru_cache(maxsize=2)
def _gumbel_noise(n, k):
    total = n * k
    idx = np.arange(total, dtype=np.uint64)
    hi = (idx >> np.uint64(32)).astype(np.uint32)
    lo = (idx & np.uint64(0xFFFFFFFF)).astype(np.uint32)
    h0, h1 = _threefry2x32(0, 42, hi, lo)
    bits = (h0 ^ h1).astype(np.uint32)
    f = ((bits >> np.uint32(9)) | np.uint32(0x3F800000)).view(np.float32)
    f = f - np.float32(1.0)
    minval, maxval = np.float32(1e-10), np.float32(1.0)
    u = np.maximum(minval, f * (maxval - minval) + minval)
    g = -np.log(-np.log(u, dtype=np.float32), dtype=np.float32)
    return g.reshape(n, k)


def _sc_assign(logits_flat, g_flat, abs_flat, n):
    k = _NUM_ABS
    try:
        info = plsc.get_sparse_core_info()
        nc, ns = info.num_cores, info.num_subcores
    except Exception:
        nc, ns = 2, 16
    nw = nc * ns
    full_chunks_last = (n - (nw - 1) * _PERW) // _CH
    tail_base = (nw - 1) * _PERW + full_chunks_last * _CH
    tail = n - tail_base

    mesh = plsc.VectorSubcoreMesh(core_axis_name="c", subcore_axis_name="s")

    @functools.partial(
        pl.kernel, mesh=mesh,
        out_type=jax.ShapeDtypeStruct((n,), jnp.float32),
        compiler_params=pltpu.CompilerParams(needs_layout_passes=False),
        scratch_types=[
            pltpu.VMEM((_CH * k,), jnp.float32),
            pltpu.VMEM((_CH * k,), jnp.float32),
            pltpu.VMEM((k,), jnp.float32),
            pltpu.VMEM((_CH,), jnp.float32),
        ],
    )
    def sc_fn(l_hbm, g_hbm, a_hbm, out_hbm, lbuf, gbuf, abuf, obuf):
        wid = lax.axis_index("s") * nc + lax.axis_index("c")
        pltpu.sync_copy(a_hbm, abuf)
        lane = lax.iota(jnp.int32, 16)
        lane26 = lane * k

        def do_chunk(base, nag):
            ngroups = nag // 16
            pltpu.sync_copy(l_hbm.at[pl.ds(base * k, nag * k)],
                            lbuf.at[pl.ds(0, nag * k)])
            pltpu.sync_copy(g_hbm.at[pl.ds(base * k, nag * k)],
                            gbuf.at[pl.ds(0, nag * k)])

            @pl.loop(0, ngroups)
            def gbody(gi):
                idx0 = lane26 + gi * (16 * k)
                best = jnp.full((16,), -jnp.inf, jnp.float32)
                bestj = jnp.zeros((16,), jnp.int32)
                for j in range(k):
                    lv = plsc.load_gather(lbuf, [idx0 + j])
                    gv = plsc.load_gather(gbuf, [idx0 + j])
                    z = lv + gv
                    better = z > best
                    best = jnp.where(better, z, best)
                    bestj = jnp.where(better,
                                      jnp.full((16,), j, jnp.int32), bestj)
                av = plsc.load_gather(abuf, [bestj])
                plsc.store_scatter(obuf, [gi * 16 + lane], av)

            pltpu.sync_copy(obuf.at[pl.ds(0, nag)],
                            out_hbm.at[pl.ds(base, nag)])

        base_w = wid * _PERW
        for c in range(_PERW // _CH):
            @pl.when(base_w + (c + 1) * _CH <= n)
            def _():
                do_chunk(base_w + c * _CH, _CH)

        if tail > 0:
            @pl.when(wid == nw - 1)
            def _():
                do_chunk(tail_base, tail)

    return sc_fn(logits_flat, g_flat, abs_flat)


def _tc_body(as_ref, emb_ref, wet_ref, w0_ref, b_ref, out_ref):
    a_col = jnp.transpose(as_ref[0], (1, 0))
    y = jnp.dot(emb_ref[...], wet_ref[...],
                preferred_element_type=jnp.float32)
    y = y + a_col * w0_ref[...] + b_ref[...]
    t = jnp.exp(y[:, 1:2] - y[:, 0:1])
    r = 1.0 / (1.0 + t)
    out_ref[...] = jnp.concatenate([r, t * r], axis=1)


def kernel(abs_actions, assigner_logits, emb_table, W, b):
    n, k = assigner_logits.shape
    d = emb_table.shape[1]
    g_flat = jnp.asarray(_gumbel_noise(n, k).reshape(-1))
    assigned = _sc_assign(assigner_logits.reshape(-1), g_flat, abs_actions, n)

    nb = n // _ROW_BLOCK
    asr = assigned.reshape(nb, 1, _ROW_BLOCK)
    wet = W[:, 1:].T
    w0 = W[:, 0].reshape(1, -1)
    b_row = b.reshape(1, -1)

    out = pl.pallas_call(
        _tc_body,
        grid=(nb,),
        in_specs=[
            pl.BlockSpec((1, 1, _ROW_BLOCK), lambda i: (i, 0, 0)),
            pl.BlockSpec((_ROW_BLOCK, d), lambda i: (i, 0)),
            pl.BlockSpec((d, W.shape[0]), lambda i: (0, 0)),
            pl.BlockSpec((1, W.shape[0]), lambda i: (0, 0)),
            pl.BlockSpec((1, W.shape[0]), lambda i: (0, 0)),
        ],
        out_specs=pl.BlockSpec((_ROW_BLOCK, W.shape[0]), lambda i: (i, 0)),
        out_shape=jax.ShapeDtypeStruct((n, W.shape[0]), jnp.float32),
        compiler_params=pltpu.CompilerParams(
            dimension_semantics=("arbitrary",)),
    )(asr, emb_table, wet, w0, b_row)
    return out


# speedup vs baseline: 1.0011x; 1.0011x over previous
import functools

import numpy as np
import jax
import jax.numpy as jnp
from jax import lax
from jax.experimental import pallas as pl
from jax.experimental.pallas import tpu as pltpu
from jax.experimental.pallas import tpu_sc as plsc

_NUM_ABS = 26
_ROW_BLOCK = 4000
_CH = 640
_PERW = 3200


def _rotl32(x, r):
    r = np.uint32(r)
    return ((x << r) | (x >> (np.uint32(32) - r))).astype(np.uint32)


def _threefry2x32(k0, k1, x0, x1):
    x0 = x0.astype(np.uint32)
    x1 = x1.astype(np.uint32)
    ks0 = np.uint32(k0)
    ks1 = np.uint32(k1)
    ks2 = np.uint32(ks0 ^ ks1 ^ np.uint32(0x1BD11BDA))
    ks = (ks0, ks1, ks2)
    rotations = ((13, 15, 26, 6), (17, 29, 16, 24))
    x0 = (x0 + ks0).astype(np.uint32)
    x1 = (x1 + ks1).astype(np.uint32)
    for i in range(5):
        for r in rotations[i % 2]:
            x0 = (x0 + x1).astype(np.uint32)
            x1 = _rotl32(x1, r)
            x1 = (x1 ^ x0).astype(np.uint32)
        x0 = (x0 + ks[(i + 1) % 3]).astype(np.uint32)
        x1 = (x1 + ks[(i + 2) % 3] + np.uint32(i + 1)).astype(np.uint32)
    return x0, x1


@functools.lru_cache(maxsize=2)
def _gumbel_noise(n, k):
    total = n * k
    idx = np.arange(total, dtype=np.uint64)
    hi = (idx >> np.uint64(32)).astype(np.uint32)
    lo = (idx & np.uint64(0xFFFFFFFF)).astype(np.uint32)
    h0, h1 = _threefry2x32(0, 42, hi, lo)
    bits = (h0 ^ h1).astype(np.uint32)
    f = ((bits >> np.uint32(9)) | np.uint32(0x3F800000)).view(np.float32)
    f = f - np.float32(1.0)
    minval, maxval = np.float32(1e-10), np.float32(1.0)
    u = np.maximum(minval, f * (maxval - minval) + minval)
    g = -np.log(-np.log(u, dtype=np.float32), dtype=np.float32)
    return g.reshape(n, k)


def _sc_assign(logits_flat, g_flat, abs_flat, n):
    k = _NUM_ABS
    try:
        info = plsc.get_sparse_core_info()
        nc, ns = info.num_cores, info.num_subcores
    except Exception:
        nc, ns = 2, 16
    nw = nc * ns
    full_chunks_last = (n - (nw - 1) * _PERW) // _CH
    tail_base = (nw - 1) * _PERW + full_chunks_last * _CH
    tail = n - tail_base

    mesh = plsc.VectorSubcoreMesh(core_axis_name="c", subcore_axis_name="s")

    @functools.partial(
        pl.kernel, mesh=mesh,
        out_type=jax.ShapeDtypeStruct((n,), jnp.float32),
        compiler_params=pltpu.CompilerParams(needs_layout_passes=False),
        scratch_types=[
            pltpu.VMEM((_CH * k,), jnp.float32),
            pltpu.VMEM((_CH * k,), jnp.float32),
            pltpu.VMEM((k,), jnp.float32),
            pltpu.VMEM((_CH,), jnp.float32),
        ],
    )
    def sc_fn(l_hbm, g_hbm, a_hbm, out_hbm, lbuf, gbuf, abuf, obuf):
        wid = lax.axis_index("s") * nc + lax.axis_index("c")
        pltpu.sync_copy(a_hbm, abuf)
        lane = lax.iota(jnp.int32, 16)
        lane26 = lane * k

        def do_chunk(base, nag):
            ngroups = nag // 16
            pltpu.sync_copy(l_hbm.at[pl.ds(base * k, nag * k)],
                            lbuf.at[pl.ds(0, nag * k)])
            pltpu.sync_copy(g_hbm.at[pl.ds(base * k, nag * k)],
                            gbuf.at[pl.ds(0, nag * k)])

            @pl.loop(0, ngroups)
            def gbody(gi):
                idx0 = lane26 + gi * (16 * k)
                best = jnp.full((16,), -jnp.inf, jnp.float32)
                bestj = jnp.zeros((16,), jnp.int32)
                for j in range(k):
                    lv = plsc.load_gather(lbuf, [idx0 + j])
                    gv = gbuf[pl.ds(gi * (16 * k) + j * 16, 16)]
                    z = lv + gv
                    better = z > best
                    best = jnp.where(better, z, best)
                    bestj = jnp.where(better,
                                      jnp.full((16,), j, jnp.int32), bestj)
                av = plsc.load_gather(abuf, [bestj])
                plsc.store_scatter(obuf, [gi * 16 + lane], av)

            pltpu.sync_copy(obuf.at[pl.ds(0, nag)],
                            out_hbm.at[pl.ds(base, nag)])

        base_w = wid * _PERW
        for c in range(_PERW // _CH):
            @pl.when(base_w + (c + 1) * _CH <= n)
            def _():
                do_chunk(base_w + c * _CH, _CH)

        if tail > 0:
            @pl.when(wid == nw - 1)
            def _():
                do_chunk(tail_base, tail)

    return sc_fn(logits_flat, g_flat, abs_flat)


def _tc_body(as_ref, emb_ref, wet_ref, w0_ref, b_ref, out_ref):
    a_col = jnp.transpose(as_ref[0], (1, 0))
    y = jnp.dot(emb_ref[...], wet_ref[...],
                preferred_element_type=jnp.float32)
    y = y + a_col * w0_ref[...] + b_ref[...]
    t = jnp.exp(y[:, 1:2] - y[:, 0:1])
    r = 1.0 / (1.0 + t)
    out_ref[...] = jnp.concatenate([r, t * r], axis=1)


def kernel(abs_actions, assigner_logits, emb_table, W, b):
    n, k = assigner_logits.shape
    d = emb_table.shape[1]
    g_perm = _gumbel_noise(n, k).reshape(n // 16, 16, k).transpose(0, 2, 1)
    g_flat = jnp.asarray(np.ascontiguousarray(g_perm).reshape(-1))
    assigned = _sc_assign(assigner_logits.reshape(-1), g_flat, abs_actions, n)

    nb = n // _ROW_BLOCK
    asr = assigned.reshape(nb, 1, _ROW_BLOCK)
    wet = W[:, 1:].T
    w0 = W[:, 0].reshape(1, -1)
    b_row = b.reshape(1, -1)

    out = pl.pallas_call(
        _tc_body,
        grid=(nb,),
        in_specs=[
            pl.BlockSpec((1, 1, _ROW_BLOCK), lambda i: (i, 0, 0)),
            pl.BlockSpec((_ROW_BLOCK, d), lambda i: (i, 0)),
            pl.BlockSpec((d, W.shape[0]), lambda i: (0, 0)),
            pl.BlockSpec((1, W.shape[0]), lambda i: (0, 0)),
            pl.BlockSpec((1, W.shape[0]), lambda i: (0, 0)),
        ],
        out_specs=pl.BlockSpec((_ROW_BLOCK, W.shape[0]), lambda i: (i, 0)),
        out_shape=jax.ShapeDtypeStruct((n, W.shape[0]), jnp.float32),
        compiler_params=pltpu.CompilerParams(
            dimension_semantics=("arbitrary",)),
    )(asr, emb_table, wet, w0, b_row)
    return out


# speedup vs baseline: 1.0224x; 1.0213x over previous
import functools

import numpy as np
import jax
import jax.numpy as jnp
from jax import lax
from jax.experimental import pallas as pl
from jax.experimental.pallas import tpu as pltpu
from jax.experimental.pallas import tpu_sc as plsc

_NUM_ABS = 26
_ROW_BLOCK = 4000
_CH = 1600
_PERW = 3200


def _rotl32(x, r):
    r = np.uint32(r)
    return ((x << r) | (x >> (np.uint32(32) - r))).astype(np.uint32)


def _threefry2x32(k0, k1, x0, x1):
    x0 = x0.astype(np.uint32)
    x1 = x1.astype(np.uint32)
    ks0 = np.uint32(k0)
    ks1 = np.uint32(k1)
    ks2 = np.uint32(ks0 ^ ks1 ^ np.uint32(0x1BD11BDA))
    ks = (ks0, ks1, ks2)
    rotations = ((13, 15, 26, 6), (17, 29, 16, 24))
    x0 = (x0 + ks0).astype(np.uint32)
    x1 = (x1 + ks1).astype(np.uint32)
    for i in range(5):
        for r in rotations[i % 2]:
            x0 = (x0 + x1).astype(np.uint32)
            x1 = _rotl32(x1, r)
            x1 = (x1 ^ x0).astype(np.uint32)
        x0 = (x0 + ks[(i + 1) % 3]).astype(np.uint32)
        x1 = (x1 + ks[(i + 2) % 3] + np.uint32(i + 1)).astype(np.uint32)
    return x0, x1


@functools.lru_cache(maxsize=2)
def _gumbel_noise(n, k):
    total = n * k
    idx = np.arange(total, dtype=np.uint64)
    hi = (idx >> np.uint64(32)).astype(np.uint32)
    lo = (idx & np.uint64(0xFFFFFFFF)).astype(np.uint32)
    h0, h1 = _threefry2x32(0, 42, hi, lo)
    bits = (h0 ^ h1).astype(np.uint32)
    f = ((bits >> np.uint32(9)) | np.uint32(0x3F800000)).view(np.float32)
    f = f - np.float32(1.0)
    minval, maxval = np.float32(1e-10), np.float32(1.0)
    u = np.maximum(minval, f * (maxval - minval) + minval)
    g = -np.log(-np.log(u, dtype=np.float32), dtype=np.float32)
    return g.reshape(n, k)


def _sc_assign(logits_flat, g_flat, abs_flat, n):
    k = _NUM_ABS
    try:
        info = plsc.get_sparse_core_info()
        nc, ns = info.num_cores, info.num_subcores
    except Exception:
        nc, ns = 2, 16
    nw = nc * ns
    full_chunks_last = (n - (nw - 1) * _PERW) // _CH
    tail_base = (nw - 1) * _PERW + full_chunks_last * _CH
    tail = n - tail_base

    mesh = plsc.VectorSubcoreMesh(core_axis_name="c", subcore_axis_name="s")

    @functools.partial(
        pl.kernel, mesh=mesh,
        out_type=jax.ShapeDtypeStruct((n,), jnp.float32),
        compiler_params=pltpu.CompilerParams(needs_layout_passes=False),
        scratch_types=[
            pltpu.VMEM((_CH * k,), jnp.float32),
            pltpu.VMEM((_CH * k,), jnp.float32),
            pltpu.VMEM((k,), jnp.float32),
            pltpu.VMEM((_CH,), jnp.float32),
        ],
    )
    def sc_fn(l_hbm, g_hbm, a_hbm, out_hbm, lbuf, gbuf, abuf, obuf):
        wid = lax.axis_index("s") * nc + lax.axis_index("c")
        pltpu.sync_copy(a_hbm, abuf)
        lane = lax.iota(jnp.int32, 16)
        lane26 = lane * k

        def do_chunk(base, nag):
            ngroups = nag // 16
            pltpu.sync_copy(l_hbm.at[pl.ds(base * k, nag * k)],
                            lbuf.at[pl.ds(0, nag * k)])
            pltpu.sync_copy(g_hbm.at[pl.ds(base * k, nag * k)],
                            gbuf.at[pl.ds(0, nag * k)])

            @pl.loop(0, ngroups)
            def gbody(gi):
                idx0 = lane26 + gi * (16 * k)
                best = jnp.full((16,), -jnp.inf, jnp.float32)
                bestj = jnp.zeros((16,), jnp.int32)
                for j in range(k):
                    lv = plsc.load_gather(lbuf, [idx0 + j])
                    gv = gbuf[pl.ds(gi * (16 * k) + j * 16, 16)]
                    z = lv + gv
                    better = z > best
                    best = jnp.where(better, z, best)
                    bestj = jnp.where(better,
                                      jnp.full((16,), j, jnp.int32), bestj)
                av = plsc.load_gather(abuf, [bestj])
                plsc.store_scatter(obuf, [gi * 16 + lane], av)

            pltpu.sync_copy(obuf.at[pl.ds(0, nag)],
                            out_hbm.at[pl.ds(base, nag)])

        base_w = wid * _PERW
        for c in range(_PERW // _CH):
            @pl.when(base_w + (c + 1) * _CH <= n)
            def _():
                do_chunk(base_w + c * _CH, _CH)

        if tail > 0:
            @pl.when(wid == nw - 1)
            def _():
                do_chunk(tail_base, tail)

    return sc_fn(logits_flat, g_flat, abs_flat)


def _tc_body(as_ref, emb_ref, wet_ref, w0_ref, b_ref, out_ref):
    a_col = jnp.transpose(as_ref[0], (1, 0))
    y = jnp.dot(emb_ref[...], wet_ref[...],
                preferred_element_type=jnp.float32)
    y = y + a_col * w0_ref[...] + b_ref[...]
    t = jnp.exp(y[:, 1:2] - y[:, 0:1])
    r = 1.0 / (1.0 + t)
    out_ref[...] = jnp.concatenate([r, t * r], axis=1)


def kernel(abs_actions, assigner_logits, emb_table, W, b):
    n, k = assigner_logits.shape
    d = emb_table.shape[1]
    g_perm = _gumbel_noise(n, k).reshape(n // 16, 16, k).transpose(0, 2, 1)
    g_flat = jnp.asarray(np.ascontiguousarray(g_perm).reshape(-1))
    assigned = _sc_assign(assigner_logits.reshape(-1), g_flat, abs_actions, n)

    nb = n // _ROW_BLOCK
    asr = assigned.reshape(nb, 1, _ROW_BLOCK)
    wet = W[:, 1:].T
    w0 = W[:, 0].reshape(1, -1)
    b_row = b.reshape(1, -1)

    out = pl.pallas_call(
        _tc_body,
        grid=(nb,),
        in_specs=[
            pl.BlockSpec((1, 1, _ROW_BLOCK), lambda i: (i, 0, 0)),
            pl.BlockSpec((_ROW_BLOCK, d), lambda i: (i, 0)),
            pl.BlockSpec((d, W.shape[0]), lambda i: (0, 0)),
            pl.BlockSpec((1, W.shape[0]), lambda i: (0, 0)),
            pl.BlockSpec((1, W.shape[0]), lambda i: (0, 0)),
        ],
        out_specs=pl.BlockSpec((_ROW_BLOCK, W.shape[0]), lambda i: (i, 0)),
        out_shape=jax.ShapeDtypeStruct((n, W.shape[0]), jnp.float32),
        compiler_params=pltpu.CompilerParams(
            dimension_semantics=("arbitrary",)),
    )(asr, emb_table, wet, w0, b_row)
    return out
